# Initial kernel scaffold; baseline (speedup 1.0000x reference)
import jax
import jax.numpy as jnp
from jax.experimental import pallas as pl


def kernel(x, train_pos_edge_index, pos_edge_index, neg_edge_index, W1, b1, W2, b2):
    raise NotImplementedError("write your pallas kernel here")


# speedup vs baseline: 11.2568x; 11.2568x over previous
import functools

import jax
import jax.numpy as jnp
from jax import lax
from jax.experimental import pallas as pl
from jax.experimental.pallas import tpu as pltpu
from jax.experimental.pallas import tpu_sc as plsc

N = 10000
D = 128
H = 50
HP = 64
Z = 16
E = 320000
NC = 2
NS = 16
NW = NC * NS
CHUNK = 128

EPAD = 323584
EW = EPAD // NW
ECH = EW // CHUNK

P = 200000
PPAD = 200704
PW = PPAD // NW
PCH = PW // CHUNK

NROWS = 10240
TRASH = N

_mesh = functools.partial(
    plsc.VectorSubcoreMesh,
    core_axis_name="c", subcore_axis_name="s", num_cores=NC, num_subcores=NS,
)


def _zero_vmem(ref, rows, width):
    z = jnp.zeros((16,), jnp.float32)
    for i in range(rows):
        for j in range(width // 16):
            ref[i, pl.ds(j * 16, 16)] = z


@functools.partial(
    pl.kernel,
    out_type=jax.ShapeDtypeStruct((NC, NROWS, 16), jnp.float32),
    mesh=_mesh(),
    compiler_params=pltpu.CompilerParams(use_tc_tiling_on_sc=False),
    scratch_types=[
        pltpu.VMEM((CHUNK,), jnp.int32),
        pltpu.VMEM((CHUNK, 16), jnp.float32),
        pltpu.VMEM((CHUNK, 16), jnp.float32),
        pltpu.VMEM_SHARED((NROWS, 16), jnp.float32),
    ],
)
def _sc_degree(dst_hbm, out_hbm, idx_v, ones_v, zero_v, acc_sh):
    c = lax.axis_index("c")
    s = lax.axis_index("s")
    wid = c * NS + s
    base = wid * EW

    one = jnp.full((16,), 1.0, jnp.float32)
    for i in range(CHUNK):
        ones_v[i, :] = one
    _zero_vmem(zero_v, CHUNK, 16)
    zrows = NROWS // NS
    for k in range(zrows // CHUNK):
        pltpu.sync_copy(zero_v, acc_sh.at[pl.ds(s * zrows + k * CHUNK, CHUNK)])
    plsc.subcore_barrier()

    def body(ci, _):
        pltpu.sync_copy(dst_hbm.at[pl.ds(base + ci * CHUNK, CHUNK)], idx_v)
        pltpu.sync_copy(ones_v, acc_sh.at[idx_v], add=True)
        return _
    lax.fori_loop(0, ECH, body, None)
    plsc.subcore_barrier()

    orows = NROWS // NS
    pltpu.sync_copy(acc_sh.at[pl.ds(s * orows, orows)],
                    out_hbm.at[c, pl.ds(s * orows, orows)])


def _make_edge_pass(W):
    @functools.partial(
        pl.kernel,
        out_type=jax.ShapeDtypeStruct((NC, NROWS, W), jnp.float32),
        mesh=_mesh(),
        compiler_params=pltpu.CompilerParams(use_tc_tiling_on_sc=False),
        scratch_types=[
            pltpu.VMEM((CHUNK,), jnp.int32),
            pltpu.VMEM((CHUNK,), jnp.int32),
            pltpu.VMEM((CHUNK, W), jnp.float32),
            pltpu.VMEM_SHARED((NROWS, W), jnp.float32),
            pltpu.SemaphoreType.DMA,
        ],
    )
    def edge_pass(src_hbm, dst_hbm, hs_hbm, out_hbm,
                  isrc_v, idst_v, rows_v, acc_sh, sem):
        c = lax.axis_index("c")
        s = lax.axis_index("s")
        wid = c * NS + s
        base = wid * EW

        _zero_vmem(rows_v, CHUNK, W)
        zrows = NROWS // NS
        for k in range(zrows // CHUNK):
            pltpu.sync_copy(rows_v,
                            acc_sh.at[pl.ds(s * zrows + k * CHUNK, CHUNK)])
        plsc.subcore_barrier()

        def body(ci, _):
            off = base + ci * CHUNK
            pltpu.sync_copy(src_hbm.at[pl.ds(off, CHUNK)], isrc_v)
            pltpu.async_copy(hs_hbm.at[isrc_v], rows_v, sem).wait()
            pltpu.sync_copy(dst_hbm.at[pl.ds(off, CHUNK)], idst_v)
            pltpu.sync_copy(rows_v, acc_sh.at[idst_v], add=True)
            return _
        lax.fori_loop(0, ECH, body, None)
        plsc.subcore_barrier()

        orows = NROWS // NS
        pltpu.sync_copy(acc_sh.at[pl.ds(s * orows, orows)],
                        out_hbm.at[c, pl.ds(s * orows, orows)])

    return edge_pass


_sc_edge_pass_h = _make_edge_pass(HP)
_sc_edge_pass_z = _make_edge_pass(Z)


@functools.partial(
    pl.kernel,
    out_type=jax.ShapeDtypeStruct((PPAD, Z), jnp.float32),
    mesh=_mesh(),
    compiler_params=pltpu.CompilerParams(use_tc_tiling_on_sc=False),
    scratch_types=[
        pltpu.VMEM((CHUNK,), jnp.int32),
        pltpu.VMEM((CHUNK,), jnp.int32),
        pltpu.VMEM((CHUNK, Z), jnp.float32),
        pltpu.VMEM((CHUNK, Z), jnp.float32),
        pltpu.SemaphoreType.DMA,
    ],
)
def _sc_decode(a_hbm, b_hbm, z_hbm, out_hbm, ia_v, ib_v, za_v, zb_v, sem):
    c = lax.axis_index("c")
    s = lax.axis_index("s")
    wid = c * NS + s
    base = wid * PW

    def body(ci, _):
        off = base + ci * CHUNK
        pltpu.sync_copy(a_hbm.at[pl.ds(off, CHUNK)], ia_v)
        pltpu.async_copy(z_hbm.at[ia_v], za_v, sem).wait()
        pltpu.sync_copy(b_hbm.at[pl.ds(off, CHUNK)], ib_v)
        pltpu.async_copy(z_hbm.at[ib_v], zb_v, sem).wait()
        for i in range(CHUNK):
            za_v[i, :] = za_v[i, :] * zb_v[i, :]
        pltpu.sync_copy(za_v, out_hbm.at[pl.ds(off, CHUNK)])
        return _
    lax.fori_loop(0, PCH, body, None)


_BR = 400


def _tc_encode1(x_ref, dega_ref, degb_ref, w1_ref, hs1_ref, dinv_ref):
    deg = dega_ref[:, 0:1] + degb_ref[:, 0:1] + 1.0
    dv = lax.rsqrt(deg)
    h = jnp.dot(x_ref[...], w1_ref[...], preferred_element_type=jnp.float32)
    hs1_ref[...] = h * dv
    dinv_ref[...] = dv


def _tc_encode2(s1a_ref, s1b_ref, hs1_ref, dinv_ref, b1_ref, w2_ref, hs2_ref):
    dv = dinv_ref[...]
    h = jax.nn.relu(dv * (s1a_ref[...] + s1b_ref[...] + hs1_ref[...])
                    + b1_ref[...])
    hs2_ref[...] = jnp.dot(h, w2_ref[...],
                           preferred_element_type=jnp.float32) * dv


def _tc_final(s2a_ref, s2b_ref, hs2_ref, dinv_ref, b2_ref, z_ref):
    z_ref[...] = (dinv_ref[...] * (s2a_ref[...] + s2b_ref[...] + hs2_ref[...])
                  + b2_ref[...])


_PBR = 2048


def _tc_rowsum(prod_ref, out_ref):
    out_ref[...] = jnp.sum(prod_ref[...], axis=1, keepdims=True)


def kernel(x, train_pos_edge_index, pos_edge_index, neg_edge_index,
           W1, b1, W2, b2):
    f32 = jnp.float32
    src = train_pos_edge_index[0]
    dst = train_pos_edge_index[1]
    npad = EPAD - E
    src_p = jnp.concatenate([src, jnp.zeros((npad,), jnp.int32)])
    dst_p = jnp.concatenate([dst, jnp.full((npad,), TRASH, jnp.int32)])

    W1p = jnp.zeros((D, HP), f32).at[:, :H].set(W1)
    b1p = jnp.zeros((1, HP), f32).at[0, :H].set(b1)
    W2p = jnp.zeros((HP, Z), f32).at[:H, :].set(W2)
    b2r = b2.reshape(1, Z)

    deg_parts = _sc_degree(dst_p)

    hs1, dinv = pl.pallas_call(
        _tc_encode1,
        grid=(N // _BR,),
        in_specs=[
            pl.BlockSpec((_BR, D), lambda i: (i, 0)),
            pl.BlockSpec((_BR, 16), lambda i: (i, 0)),
            pl.BlockSpec((_BR, 16), lambda i: (i, 0)),
            pl.BlockSpec((D, HP), lambda i: (0, 0)),
        ],
        out_specs=[
            pl.BlockSpec((_BR, HP), lambda i: (i, 0)),
            pl.BlockSpec((_BR, 1), lambda i: (i, 0)),
        ],
        out_shape=[
            jax.ShapeDtypeStruct((N, HP), f32),
            jax.ShapeDtypeStruct((N, 1), f32),
        ],
    )(x, deg_parts[0], deg_parts[1], W1p)

    s1_parts = _sc_edge_pass_h(src_p, dst_p, hs1)

    hs2 = pl.pallas_call(
        _tc_encode2,
        grid=(N // _BR,),
        in_specs=[
            pl.BlockSpec((_BR, HP), lambda i: (i, 0)),
            pl.BlockSpec((_BR, HP), lambda i: (i, 0)),
            pl.BlockSpec((_BR, HP), lambda i: (i, 0)),
            pl.BlockSpec((_BR, 1), lambda i: (i, 0)),
            pl.BlockSpec((1, HP), lambda i: (0, 0)),
            pl.BlockSpec((HP, Z), lambda i: (0, 0)),
        ],
        out_specs=pl.BlockSpec((_BR, Z), lambda i: (i, 0)),
        out_shape=jax.ShapeDtypeStruct((N, Z), f32),
    )(s1_parts[0], s1_parts[1], hs1, dinv, b1p, W2p)

    s2_parts = _sc_edge_pass_z(src_p, dst_p, hs2)

    z = pl.pallas_call(
        _tc_final,
        grid=(N // _BR,),
        in_specs=[
            pl.BlockSpec((_BR, Z), lambda i: (i, 0)),
            pl.BlockSpec((_BR, Z), lambda i: (i, 0)),
            pl.BlockSpec((_BR, Z), lambda i: (i, 0)),
            pl.BlockSpec((_BR, 1), lambda i: (i, 0)),
            pl.BlockSpec((1, Z), lambda i: (0, 0)),
        ],
        out_specs=pl.BlockSpec((_BR, Z), lambda i: (i, 0)),
        out_shape=jax.ShapeDtypeStruct((N, Z), f32),
    )(s2_parts[0], s2_parts[1], hs2, dinv, b2r)

    ppad = PPAD - P
    a_idx = jnp.concatenate([pos_edge_index[0], neg_edge_index[0],
                             jnp.zeros((ppad,), jnp.int32)])
    b_idx = jnp.concatenate([pos_edge_index[1], neg_edge_index[1],
                             jnp.zeros((ppad,), jnp.int32)])
    prod = _sc_decode(a_idx, b_idx, z)

    logits_pad = pl.pallas_call(
        _tc_rowsum,
        grid=(PPAD // _PBR,),
        in_specs=[pl.BlockSpec((_PBR, Z), lambda i: (i, 0))],
        out_specs=pl.BlockSpec((_PBR, 1), lambda i: (i, 0)),
        out_shape=jax.ShapeDtypeStruct((PPAD, 1), jnp.float32),
    )(prod)
    return logits_pad[:P, 0]


# speedup vs baseline: 13.9015x; 1.2349x over previous
import functools

import jax
import jax.numpy as jnp
from jax import lax
from jax.experimental import pallas as pl
from jax.experimental.pallas import tpu as pltpu
from jax.experimental.pallas import tpu_sc as plsc

N = 10000
D = 128
H = 50
HP = 64
Z = 16
E = 320000
NC = 2
NS = 16
NW = NC * NS
CHUNK = 128

ECH = 80
EW = ECH * CHUNK
EPAD = EW * NW

P = 200000
PCH = 50
PW = PCH * CHUNK
PPAD = PW * NW

NROWS = 10240
TRASH = N

NBUF = 4
DBUF = 2

_mesh = functools.partial(
    plsc.VectorSubcoreMesh,
    core_axis_name="c", subcore_axis_name="s", num_cores=NC, num_subcores=NS,
)
_sc_params = pltpu.CompilerParams(use_tc_tiling_on_sc=False)


def _zero_vmem(ref, rows, width):
    z = jnp.zeros((16,), jnp.float32)
    for i in range(rows):
        for j in range(width // 16):
            ref[i, pl.ds(j * 16, 16)] = z


@functools.partial(
    pl.kernel,
    out_type=jax.ShapeDtypeStruct((NC, NROWS, 16), jnp.float32),
    mesh=_mesh(),
    compiler_params=_sc_params,
    scratch_types=[
        pltpu.VMEM((ECH, CHUNK), jnp.int32),
        pltpu.VMEM((CHUNK, 16), jnp.float32),
        pltpu.VMEM((CHUNK, 16), jnp.float32),
        pltpu.VMEM_SHARED((NROWS, 16), jnp.float32),
        pltpu.SemaphoreType.DMA,
        pltpu.SemaphoreType.DMA,
    ],
)
def _sc_degree(dst_hbm, out_hbm, idx_v, ones_v, zero_v, acc_sh, sem_s, sem_z):
    c = lax.axis_index("c")
    s = lax.axis_index("s")
    wid = c * NS + s

    one = jnp.full((16,), 1.0, jnp.float32)
    for i in range(CHUNK):
        ones_v[i, :] = one
    _zero_vmem(zero_v, CHUNK, 16)
    zrows = NROWS // NS
    nz = zrows // CHUNK
    for k in range(nz):
        pltpu.async_copy(zero_v,
                         acc_sh.at[pl.ds(s * zrows + k * CHUNK, CHUNK)], sem_z)
    pltpu.sync_copy(dst_hbm.at[pl.ds(wid * ECH, ECH)], idx_v)
    for k in range(nz):
        pltpu.make_async_copy(
            zero_v, acc_sh.at[pl.ds(s * zrows + k * CHUNK, CHUNK)],
            sem_z).wait()
    plsc.subcore_barrier()

    GB = 8
    def body(g, _):
        for b in range(GB):
            ci = g * GB + b
            pltpu.async_copy(ones_v, acc_sh.at[idx_v.at[ci]], sem_s, add=True)
        for b in range(GB):
            ci = g * GB + b
            pltpu.make_async_copy(ones_v, acc_sh.at[idx_v.at[ci]],
                                  sem_s).wait()
        return _
    lax.fori_loop(0, ECH // GB, body, None)
    plsc.subcore_barrier()

    orows = NROWS // NS
    pltpu.sync_copy(acc_sh.at[pl.ds(s * orows, orows)],
                    out_hbm.at[c, pl.ds(s * orows, orows)])


def _make_edge_pass(W):
    @functools.partial(
        pl.kernel,
        out_type=jax.ShapeDtypeStruct((NC, NROWS, W), jnp.float32),
        mesh=_mesh(),
        compiler_params=_sc_params,
        scratch_types=(
            [pltpu.VMEM((ECH, CHUNK), jnp.int32),
             pltpu.VMEM((ECH, CHUNK), jnp.int32)]
            + [pltpu.VMEM((CHUNK, W), jnp.float32) for _ in range(NBUF)]
            + [pltpu.VMEM_SHARED((NROWS, W), jnp.float32)]
            + [pltpu.SemaphoreType.DMA for _ in range(2 * NBUF)]
        ),
    )
    def edge_pass(src_hbm, dst_hbm, hs_hbm, out_hbm, isrc_v, idst_v, *bufs):
        rows = bufs[:NBUF]
        acc_sh = bufs[NBUF]
        sem_g = bufs[NBUF + 1:2 * NBUF + 1]
        sem_s = bufs[2 * NBUF + 1:]
        c = lax.axis_index("c")
        s = lax.axis_index("s")
        wid = c * NS + s

        pltpu.sync_copy(src_hbm.at[pl.ds(wid * ECH, ECH)], isrc_v)
        pltpu.sync_copy(dst_hbm.at[pl.ds(wid * ECH, ECH)], idst_v)

        _zero_vmem(rows[0], CHUNK, W)
        zrows = NROWS // NS
        nz = zrows // CHUNK
        for k in range(nz):
            pltpu.async_copy(
                rows[0], acc_zslice(acc_sh, s, zrows, k), sem_s[0])
        for k in range(nz):
            pltpu.make_async_copy(
                rows[0], acc_zslice(acc_sh, s, zrows, k), sem_s[0]).wait()
        plsc.subcore_barrier()

        for b in range(NBUF):
            pltpu.async_copy(hs_hbm.at[isrc_v.at[b]], rows[b], sem_g[b])

        def body(g, _):
            for b in range(NBUF):
                ci = g * NBUF + b
                pltpu.make_async_copy(hs_hbm.at[isrc_v.at[ci]], rows[b],
                                      sem_g[b]).wait()
                pltpu.async_copy(rows[b], acc_sh.at[idst_v.at[ci]],
                                 sem_s[b], add=True)
            for b in range(NBUF):
                ci = g * NBUF + b
                pltpu.make_async_copy(rows[b], acc_sh.at[idst_v.at[ci]],
                                      sem_s[b]).wait()
                nci = ci + NBUF

                @pl.when(nci < ECH)
                def _issue():
                    pltpu.async_copy(hs_hbm.at[isrc_v.at[nci]], rows[b],
                                     sem_g[b])
            return _
        lax.fori_loop(0, ECH // NBUF, body, None)
        plsc.subcore_barrier()

        orows = NROWS // NS
        pltpu.sync_copy(acc_sh.at[pl.ds(s * orows, orows)],
                        out_hbm.at[c, pl.ds(s * orows, orows)])

    return edge_pass


def acc_zslice(acc_sh, s, zrows, k):
    return acc_sh.at[pl.ds(s * zrows + k * CHUNK, CHUNK)]


_sc_edge_pass_h = _make_edge_pass(HP)
_sc_edge_pass_z = _make_edge_pass(Z)


@functools.partial(
    pl.kernel,
    out_type=jax.ShapeDtypeStruct((PPAD, Z), jnp.float32),
    mesh=_mesh(),
    compiler_params=_sc_params,
    scratch_types=(
        [pltpu.VMEM((PCH, CHUNK), jnp.int32),
         pltpu.VMEM((PCH, CHUNK), jnp.int32)]
        + [pltpu.VMEM((CHUNK, Z), jnp.float32) for _ in range(2 * DBUF)]
        + [pltpu.SemaphoreType.DMA for _ in range(3 * DBUF)]
    ),
)
def _sc_decode(a_hbm, b_hbm, z_hbm, out_hbm, ia_v, ib_v, *bufs):
    za = bufs[:DBUF]
    zb = bufs[DBUF:2 * DBUF]
    sem_a = bufs[2 * DBUF:2 * DBUF + DBUF]
    sem_b = bufs[2 * DBUF + DBUF:2 * DBUF + 2 * DBUF]
    sem_o = bufs[2 * DBUF + 2 * DBUF:]
    c = lax.axis_index("c")
    s = lax.axis_index("s")
    wid = c * NS + s
    base = wid * PW

    pltpu.sync_copy(a_hbm.at[pl.ds(wid * PCH, PCH)], ia_v)
    pltpu.sync_copy(b_hbm.at[pl.ds(wid * PCH, PCH)], ib_v)

    for b in range(DBUF):
        pltpu.async_copy(z_hbm.at[ia_v.at[b]], za[b], sem_a[b])
        pltpu.async_copy(z_hbm.at[ib_v.at[b]], zb[b], sem_b[b])

    def body(g, _):
        for b in range(DBUF):
            ci = g * DBUF + b
            off = base + ci * CHUNK
            pltpu.make_async_copy(z_hbm.at[ia_v.at[ci]], za[b],
                                  sem_a[b]).wait()
            pltpu.make_async_copy(z_hbm.at[ib_v.at[ci]], zb[b],
                                  sem_b[b]).wait()
            for i in range(CHUNK):
                za[b][i, :] = za[b][i, :] * zb[b][i, :]
            pltpu.async_copy(za[b], out_hbm.at[pl.ds(off, CHUNK)], sem_o[b])
        for b in range(DBUF):
            ci = g * DBUF + b
            off = base + ci * CHUNK
            pltpu.make_async_copy(za[b], out_hbm.at[pl.ds(off, CHUNK)],
                                  sem_o[b]).wait()
            nci = ci + DBUF

            @pl.when(nci < PCH)
            def _issue():
                pltpu.async_copy(z_hbm.at[ia_v.at[nci]], za[b], sem_a[b])
                pltpu.async_copy(z_hbm.at[ib_v.at[nci]], zb[b], sem_b[b])
        return _
    lax.fori_loop(0, PCH // DBUF, body, None)


_BR = 400


def _tc_encode1(x_ref, dega_ref, degb_ref, w1_ref, hs1_ref, dinv_ref):
    deg = dega_ref[:, 0:1] + degb_ref[:, 0:1] + 1.0
    dv = lax.rsqrt(deg)
    h = jnp.dot(x_ref[...], w1_ref[...], preferred_element_type=jnp.float32)
    hs1_ref[...] = h * dv
    dinv_ref[...] = dv


def _tc_encode2(s1a_ref, s1b_ref, hs1_ref, dinv_ref, b1_ref, w2_ref, hs2_ref):
    dv = dinv_ref[...]
    h = jax.nn.relu(dv * (s1a_ref[...] + s1b_ref[...] + hs1_ref[...])
                    + b1_ref[...])
    hs2_ref[...] = jnp.dot(h, w2_ref[...],
                           preferred_element_type=jnp.float32) * dv


def _tc_final(s2a_ref, s2b_ref, hs2_ref, dinv_ref, b2_ref, z_ref):
    z_ref[...] = (dinv_ref[...] * (s2a_ref[...] + s2b_ref[...] + hs2_ref[...])
                  + b2_ref[...])


_PBR = 2048


def _tc_rowsum(prod_ref, out_ref):
    out_ref[...] = jnp.sum(prod_ref[...], axis=1, keepdims=True)


def kernel(x, train_pos_edge_index, pos_edge_index, neg_edge_index,
           W1, b1, W2, b2):
    f32 = jnp.float32
    src = train_pos_edge_index[0]
    dst = train_pos_edge_index[1]
    npad = EPAD - E
    src_m = jnp.concatenate([src, jnp.zeros((npad,), jnp.int32)]
                            ).reshape(EPAD // CHUNK, CHUNK)
    dst_m = jnp.concatenate([dst, jnp.full((npad,), TRASH, jnp.int32)]
                            ).reshape(EPAD // CHUNK, CHUNK)

    W1p = jnp.zeros((D, HP), f32).at[:, :H].set(W1)
    b1p = jnp.zeros((1, HP), f32).at[0, :H].set(b1)
    W2p = jnp.zeros((HP, Z), f32).at[:H, :].set(W2)
    b2r = b2.reshape(1, Z)

    deg_parts = _sc_degree(dst_m)

    hs1, dinv = pl.pallas_call(
        _tc_encode1,
        grid=(N // _BR,),
        in_specs=[
            pl.BlockSpec((_BR, D), lambda i: (i, 0)),
            pl.BlockSpec((_BR, 16), lambda i: (i, 0)),
            pl.BlockSpec((_BR, 16), lambda i: (i, 0)),
            pl.BlockSpec((D, HP), lambda i: (0, 0)),
        ],
        out_specs=[
            pl.BlockSpec((_BR, HP), lambda i: (i, 0)),
            pl.BlockSpec((_BR, 1), lambda i: (i, 0)),
        ],
        out_shape=[
            jax.ShapeDtypeStruct((N, HP), f32),
            jax.ShapeDtypeStruct((N, 1), f32),
        ],
    )(x, deg_parts[0], deg_parts[1], W1p)

    s1_parts = _sc_edge_pass_h(src_m, dst_m, hs1)

    hs2 = pl.pallas_call(
        _tc_encode2,
        grid=(N // _BR,),
        in_specs=[
            pl.BlockSpec((_BR, HP), lambda i: (i, 0)),
            pl.BlockSpec((_BR, HP), lambda i: (i, 0)),
            pl.BlockSpec((_BR, HP), lambda i: (i, 0)),
            pl.BlockSpec((_BR, 1), lambda i: (i, 0)),
            pl.BlockSpec((1, HP), lambda i: (0, 0)),
            pl.BlockSpec((HP, Z), lambda i: (0, 0)),
        ],
        out_specs=pl.BlockSpec((_BR, Z), lambda i: (i, 0)),
        out_shape=jax.ShapeDtypeStruct((N, Z), f32),
    )(s1_parts[0], s1_parts[1], hs1, dinv, b1p, W2p)

    s2_parts = _sc_edge_pass_z(src_m, dst_m, hs2)

    z = pl.pallas_call(
        _tc_final,
        grid=(N // _BR,),
        in_specs=[
            pl.BlockSpec((_BR, Z), lambda i: (i, 0)),
            pl.BlockSpec((_BR, Z), lambda i: (i, 0)),
            pl.BlockSpec((_BR, Z), lambda i: (i, 0)),
            pl.BlockSpec((_BR, 1), lambda i: (i, 0)),
            pl.BlockSpec((1, Z), lambda i: (0, 0)),
        ],
        out_specs=pl.BlockSpec((_BR, Z), lambda i: (i, 0)),
        out_shape=jax.ShapeDtypeStruct((N, Z), f32),
    )(s2_parts[0], s2_parts[1], hs2, dinv, b2r)

    ppad = PPAD - P
    a_m = jnp.concatenate([pos_edge_index[0], neg_edge_index[0],
                           jnp.zeros((ppad,), jnp.int32)]
                          ).reshape(PPAD // CHUNK, CHUNK)
    b_m = jnp.concatenate([pos_edge_index[1], neg_edge_index[1],
                           jnp.zeros((ppad,), jnp.int32)]
                          ).reshape(PPAD // CHUNK, CHUNK)
    prod = _sc_decode(a_m, b_m, z)

    logits_pad = pl.pallas_call(
        _tc_rowsum,
        grid=(PPAD // _PBR,),
        in_specs=[pl.BlockSpec((_PBR, Z), lambda i: (i, 0))],
        out_specs=pl.BlockSpec((_PBR, 1), lambda i: (i, 0)),
        out_shape=jax.ShapeDtypeStruct((PPAD, 1), jnp.float32),
    )(prod)
    return logits_pad[:P, 0]
